# baseline (speedup 1.0000x reference)
import functools

import jax
import jax.numpy as jnp
from jax import lax
from jax.experimental import pallas as pl
from jax.experimental.pallas import tpu as pltpu
from jax.experimental.pallas import tpu_sc as plsc

N_NODES = 10000
N_EDGES = 320000
D = 128

NC = 2
NS = 16
NW = NC * NS
L = 16

N_PAD = 10240
ROWS = N_PAD // L
E_PER_TILE = N_EDGES // NW
CHUNK = 4000
NCHUNK = N_EDGES // CHUNK
UNROLL = 25
F_PER_TILE = D // NW

_mesh = plsc.VectorSubcoreMesh(
    core_axis_name="c", subcore_axis_name="s", num_cores=NC, num_subcores=NS)


def _tc_body(x_ref, w_ref, asrc_ref, adst_ref, h_ref, s_ref, d_ref):
    h = jnp.dot(x_ref[...], w_ref[...], preferred_element_type=jnp.float32)
    h_ref[...] = h.T
    s_ref[...] = jnp.sum(h * asrc_ref[...], axis=1, keepdims=True)
    d_ref[...] = jnp.sum(h * adst_ref[...], axis=1, keepdims=True)


def _tc_transform(x, W, a_src, a_dst):
    return pl.pallas_call(
        _tc_body,
        out_shape=[
            jax.ShapeDtypeStruct((D, N_NODES), jnp.float32),
            jax.ShapeDtypeStruct((N_NODES, 1), jnp.float32),
            jax.ShapeDtypeStruct((N_NODES, 1), jnp.float32),
        ],
    )(x, W, a_src.reshape(1, D), a_dst.reshape(1, D))


@functools.partial(
    pl.kernel,
    out_type=[
        jax.ShapeDtypeStruct((N_EDGES,), jnp.float32),
        jax.ShapeDtypeStruct((NC, ROWS, L), jnp.float32),
    ],
    mesh=_mesh,
    scratch_types=[
        pltpu.VMEM((N_NODES,), jnp.float32),
        pltpu.VMEM((N_NODES,), jnp.float32),
        pltpu.VMEM((E_PER_TILE,), jnp.int32),
        pltpu.VMEM((E_PER_TILE,), jnp.int32),
        pltpu.VMEM((E_PER_TILE,), jnp.float32),
        pltpu.VMEM((ROWS, L), jnp.float32),
        pltpu.VMEM((ROWS,), jnp.int32),
        pltpu.VMEM_SHARED((ROWS, L), jnp.float32),
    ],
    compiler_params=pltpu.CompilerParams(needs_layout_passes=False, use_tc_tiling_on_sc=False),
)
def _sc_edge(src_hbm, dst_hbm, s_hbm, d_hbm, ee_hbm, dpart_hbm,
             s_v, d_v, src_v, dst_v, ee_v, dloc_v, idx_v, dsh):
    c = lax.axis_index("c")
    s = lax.axis_index("s")
    w = s * NC + c
    base = w * E_PER_TILE

    pltpu.sync_copy(s_hbm, s_v)
    pltpu.sync_copy(d_hbm, d_v)
    pltpu.sync_copy(src_hbm.at[pl.ds(base, E_PER_TILE)], src_v)
    pltpu.sync_copy(dst_hbm.at[pl.ds(base, E_PER_TILE)], dst_v)

    zf = jnp.zeros((L,), jnp.float32)
    lane = lax.iota(jnp.int32, L)

    @plsc.parallel_loop(0, ROWS, 1, unroll=8)
    def _zero(i):
        dloc_v[i, :] = zf
        idx_v[pl.ds(i * L, L)] = lane + i * L

    @pl.when(s == 0)
    def _():
        pltpu.sync_copy(dloc_v, dsh)
    plsc.subcore_barrier()

    @plsc.parallel_loop(0, E_PER_TILE // L, 1, unroll=5)
    def _edge(i):
        sl = pl.ds(i * L, L)
        src16 = src_v[sl]
        dst16 = dst_v[sl]
        sv = plsc.load_gather(s_v, [src16])
        dv = plsc.load_gather(d_v, [dst16])
        e = sv + dv
        e = jnp.maximum(e, 0.2 * e)
        ee = jnp.exp(e)
        ee_v[sl] = ee
        plsc.addupdate_scatter(
            dloc_v, [lax.shift_right_logical(dst16, 4), dst16 & 15], ee)

    pltpu.sync_copy(ee_v, ee_hbm.at[pl.ds(base, E_PER_TILE)])

    pltpu.sync_copy(dloc_v, dsh.at[idx_v], add=True)
    plsc.subcore_barrier()

    @pl.when(s == 0)
    def _():
        pltpu.sync_copy(dsh, dloc_v)
        pltpu.sync_copy(dloc_v, dpart_hbm.at[c])


@functools.partial(
    pl.kernel,
    out_type=jax.ShapeDtypeStruct((NW, F_PER_TILE, N_NODES), jnp.float32),
    mesh=_mesh,
    scratch_types=[
        pltpu.VMEM((F_PER_TILE, N_NODES), jnp.float32),
        pltpu.VMEM((F_PER_TILE, N_NODES), jnp.float32),
        pltpu.VMEM((N_PAD,), jnp.float32),
        pltpu.VMEM((N_PAD,), jnp.float32),
        pltpu.VMEM((CHUNK,), jnp.int32),
        pltpu.VMEM((CHUNK,), jnp.int32),
        pltpu.VMEM((CHUNK,), jnp.float32),
        pltpu.VMEM((CHUNK,), jnp.int32),
        pltpu.VMEM((CHUNK,), jnp.int32),
        pltpu.VMEM((CHUNK,), jnp.float32),
        pltpu.SemaphoreType.DMA,
        pltpu.SemaphoreType.DMA,
    ],
    compiler_params=pltpu.CompilerParams(needs_layout_passes=False, use_tc_tiling_on_sc=False),
)
def _sc_spmm(ht_hbm, src_hbm, dst_hbm, ee_hbm, dpart_hbm, out_hbm,
             h_v, out_v, inv_v, tmp_v, srcA, dstA, eeA, srcB, dstB, eeB,
             semA, semB):
    c = lax.axis_index("c")
    s = lax.axis_index("s")
    w = s * NC + c

    pltpu.sync_copy(ht_hbm.at[w], h_v)
    pltpu.sync_copy(dpart_hbm.at[0], inv_v)
    pltpu.sync_copy(dpart_hbm.at[1], tmp_v)

    zf = jnp.zeros((L,), jnp.float32)

    @plsc.parallel_loop(0, ROWS, 1, unroll=8)
    def _prep(i):
        sl = pl.ds(i * L, L)
        t = inv_v[sl] + tmp_v[sl]
        inv_v[sl] = 1.0 / (t + 1e-16)

    for f in range(F_PER_TILE):
        @plsc.parallel_loop(0, N_NODES // L, 1, unroll=5)
        def _zero(i, f=f):
            out_v[f, pl.ds(i * L, L)] = zf

    fconsts = [jnp.full((L,), f, jnp.int32) for f in range(F_PER_TILE)]

    def _start(off, sbuf, dbuf, ebuf, sem):
        pltpu.async_copy(src_hbm.at[pl.ds(off, CHUNK)], sbuf, sem)
        pltpu.async_copy(dst_hbm.at[pl.ds(off, CHUNK)], dbuf, sem)
        pltpu.async_copy(ee_hbm.at[pl.ds(off, CHUNK)], ebuf, sem)

    def _drain(sbuf, dbuf, ebuf, sem):
        pltpu.make_async_copy(src_hbm.at[pl.ds(0, CHUNK)], sbuf, sem).wait()
        pltpu.make_async_copy(dst_hbm.at[pl.ds(0, CHUNK)], dbuf, sem).wait()
        pltpu.make_async_copy(ee_hbm.at[pl.ds(0, CHUNK)], ebuf, sem).wait()

    def _process(sbuf, dbuf, ebuf):
        @plsc.parallel_loop(0, CHUNK // L, 1, unroll=UNROLL)
        def _grp(i):
            sl = pl.ds(i * L, L)
            src16 = sbuf[sl]
            dst16 = dbuf[sl]
            w16 = ebuf[sl]
            for f in range(F_PER_TILE):
                g = plsc.load_gather(h_v, [fconsts[f], src16])
                plsc.addupdate_scatter(out_v, [fconsts[f], dst16], g * w16)

    _start(0, srcA, dstA, eeA, semA)

    def _pair(ci, carry):
        _start((2 * ci + 1) * CHUNK, srcB, dstB, eeB, semB)
        _drain(srcA, dstA, eeA, semA)
        _process(srcA, dstA, eeA)
        off_next = jnp.minimum(2 * ci + 2, NCHUNK - 1) * CHUNK
        _start(off_next, srcA, dstA, eeA, semA)
        _drain(srcB, dstB, eeB, semB)
        _process(srcB, dstB, eeB)
        return carry
    lax.fori_loop(0, NCHUNK // 2, _pair, 0)
    _drain(srcA, dstA, eeA, semA)

    @plsc.parallel_loop(0, N_NODES // L, 1, unroll=5)
    def _scale(i):
        sl = pl.ds(i * L, L)
        iv = inv_v[sl]
        for f in range(F_PER_TILE):
            out_v[f, sl] = out_v[f, sl] * iv

    pltpu.sync_copy(out_v, out_hbm.at[w])


def kernel(x, edge_index, W, a_src, a_dst):
    src = edge_index[0]
    dst = edge_index[1]
    h_T, s2, d2 = _tc_transform(x, W, a_src, a_dst)
    s = s2.reshape(N_NODES)
    d = d2.reshape(N_NODES)
    ee, dpart = _sc_edge(src, dst, s, d)
    h_t = h_T.reshape(NW, F_PER_TILE, N_NODES)
    out_t = _sc_spmm(h_t, src, dst, ee, dpart.reshape(NC, N_PAD))
    return out_t.reshape(D, N_NODES).T


# speedup vs baseline: 1.2857x; 1.2857x over previous
import functools

import jax
import jax.numpy as jnp
from jax import lax
from jax.experimental import pallas as pl
from jax.experimental.pallas import tpu as pltpu
from jax.experimental.pallas import tpu_sc as plsc

N_NODES = 10000
N_EDGES = 320000
D = 128

NC = 2
NS = 16
NW = NC * NS
L = 16

N_PAD = 10240
ROWS = N_PAD // L
E_PER_TILE = N_EDGES // NW
CHUNK = 3200
NCHUNK = N_EDGES // CHUNK
UNROLL = 10
F_PER_TILE = D // NW

_mesh = plsc.VectorSubcoreMesh(
    core_axis_name="c", subcore_axis_name="s", num_cores=NC, num_subcores=NS)


def _tc_sd_body(x_ref, w_ref, a2_ref, s_ref, d_ref):
    wa = jnp.dot(w_ref[...], a2_ref[...], preferred_element_type=jnp.float32)
    sd = jnp.dot(x_ref[...], wa, preferred_element_type=jnp.float32)
    s_ref[...] = sd[:, 0:1]
    d_ref[...] = sd[:, 1:2]


def _tc_sd(x, W, a2):
    return pl.pallas_call(
        _tc_sd_body,
        out_shape=[
            jax.ShapeDtypeStruct((N_NODES, 1), jnp.float32),
            jax.ShapeDtypeStruct((N_NODES, 1), jnp.float32),
        ],
    )(x, W, a2)


def _tc_h_body(x_ref, w_ref, h_ref):
    h = jnp.dot(x_ref[...], w_ref[...], preferred_element_type=jnp.float32)
    h_ref[...] = h.T


def _tc_h(x, W):
    return pl.pallas_call(
        _tc_h_body,
        out_shape=jax.ShapeDtypeStruct((D, N_NODES), jnp.float32),
    )(x, W)


@functools.partial(
    pl.kernel,
    out_type=[
        jax.ShapeDtypeStruct((N_EDGES,), jnp.float32),
        jax.ShapeDtypeStruct((NC, ROWS, L), jnp.float32),
    ],
    mesh=_mesh,
    scratch_types=[
        pltpu.VMEM((N_NODES,), jnp.float32),
        pltpu.VMEM((N_NODES,), jnp.float32),
        pltpu.VMEM((E_PER_TILE,), jnp.int32),
        pltpu.VMEM((E_PER_TILE,), jnp.int32),
        pltpu.VMEM((E_PER_TILE,), jnp.float32),
        pltpu.VMEM((ROWS, L), jnp.float32),
        pltpu.VMEM((ROWS,), jnp.int32),
        pltpu.VMEM_SHARED((ROWS, L), jnp.float32),
    ],
    compiler_params=pltpu.CompilerParams(needs_layout_passes=False, use_tc_tiling_on_sc=False),
)
def _sc_edge(src_hbm, dst_hbm, s_hbm, d_hbm, ee_hbm, dpart_hbm,
             s_v, d_v, src_v, dst_v, ee_v, dloc_v, idx_v, dsh):
    c = lax.axis_index("c")
    s = lax.axis_index("s")
    w = s * NC + c
    base = w * E_PER_TILE

    pltpu.sync_copy(s_hbm, s_v)
    pltpu.sync_copy(d_hbm, d_v)
    pltpu.sync_copy(src_hbm.at[pl.ds(base, E_PER_TILE)], src_v)
    pltpu.sync_copy(dst_hbm.at[pl.ds(base, E_PER_TILE)], dst_v)

    zf = jnp.zeros((L,), jnp.float32)
    lane = lax.iota(jnp.int32, L)

    @plsc.parallel_loop(0, ROWS, 1, unroll=8)
    def _zero(i):
        dloc_v[i, :] = zf
        idx_v[pl.ds(i * L, L)] = lane + i * L

    @pl.when(s == 0)
    def _():
        pltpu.sync_copy(dloc_v, dsh)
    plsc.subcore_barrier()

    @plsc.parallel_loop(0, E_PER_TILE // L, 1, unroll=5)
    def _edge(i):
        sl = pl.ds(i * L, L)
        src16 = src_v[sl]
        dst16 = dst_v[sl]
        sv = plsc.load_gather(s_v, [src16])
        dv = plsc.load_gather(d_v, [dst16])
        e = sv + dv
        e = jnp.maximum(e, 0.2 * e)
        ee = jnp.exp(e)
        ee_v[sl] = ee
        plsc.addupdate_scatter(
            dloc_v, [lax.shift_right_logical(dst16, 4), dst16 & 15], ee)

    pltpu.sync_copy(ee_v, ee_hbm.at[pl.ds(base, E_PER_TILE)])

    pltpu.sync_copy(dloc_v, dsh.at[idx_v], add=True)
    plsc.subcore_barrier()

    @pl.when(s == 0)
    def _():
        pltpu.sync_copy(dsh, dloc_v)
        pltpu.sync_copy(dloc_v, dpart_hbm.at[c])


@functools.partial(
    pl.kernel,
    out_type=jax.ShapeDtypeStruct((NW, F_PER_TILE, N_NODES), jnp.float32),
    mesh=_mesh,
    scratch_types=[
        pltpu.VMEM((F_PER_TILE, N_NODES), jnp.float32),
        pltpu.VMEM((F_PER_TILE, N_NODES), jnp.float32),
        pltpu.VMEM((N_PAD,), jnp.float32),
        pltpu.VMEM((N_PAD,), jnp.float32),
        pltpu.VMEM((CHUNK,), jnp.int32),
        pltpu.VMEM((CHUNK,), jnp.int32),
        pltpu.VMEM((CHUNK,), jnp.float32),
        pltpu.VMEM((CHUNK,), jnp.int32),
        pltpu.VMEM((CHUNK,), jnp.int32),
        pltpu.VMEM((CHUNK,), jnp.float32),
        pltpu.SemaphoreType.DMA,
        pltpu.SemaphoreType.DMA,
    ],
    compiler_params=pltpu.CompilerParams(needs_layout_passes=False, use_tc_tiling_on_sc=False),
)
def _sc_spmm(ht_hbm, src_hbm, dst_hbm, ee_hbm, dpart_hbm, out_hbm,
             h_v, out_v, inv_v, tmp_v, srcA, dstA, eeA, srcB, dstB, eeB,
             semA, semB):
    c = lax.axis_index("c")
    s = lax.axis_index("s")
    w = s * NC + c

    pltpu.sync_copy(ht_hbm.at[w], h_v)
    pltpu.sync_copy(dpart_hbm.at[0], inv_v)
    pltpu.sync_copy(dpart_hbm.at[1], tmp_v)

    zf = jnp.zeros((L,), jnp.float32)

    @plsc.parallel_loop(0, ROWS, 1, unroll=8)
    def _prep(i):
        sl = pl.ds(i * L, L)
        t = inv_v[sl] + tmp_v[sl]
        inv_v[sl] = 1.0 / (t + 1e-16)

    for f in range(F_PER_TILE):
        @plsc.parallel_loop(0, N_NODES // L, 1, unroll=5)
        def _zero(i, f=f):
            out_v[f, pl.ds(i * L, L)] = zf

    fconsts = [jnp.full((L,), f, jnp.int32) for f in range(F_PER_TILE)]

    def _start(off, sbuf, dbuf, ebuf, sem):
        pltpu.async_copy(src_hbm.at[pl.ds(off, CHUNK)], sbuf, sem)
        pltpu.async_copy(dst_hbm.at[pl.ds(off, CHUNK)], dbuf, sem)
        pltpu.async_copy(ee_hbm.at[pl.ds(off, CHUNK)], ebuf, sem)

    def _drain(sbuf, dbuf, ebuf, sem):
        pltpu.make_async_copy(src_hbm.at[pl.ds(0, CHUNK)], sbuf, sem).wait()
        pltpu.make_async_copy(dst_hbm.at[pl.ds(0, CHUNK)], dbuf, sem).wait()
        pltpu.make_async_copy(ee_hbm.at[pl.ds(0, CHUNK)], ebuf, sem).wait()

    def _process(sbuf, dbuf, ebuf):
        @plsc.parallel_loop(0, CHUNK // L, 1, unroll=UNROLL)
        def _grp(i):
            sl = pl.ds(i * L, L)
            src16 = sbuf[sl]
            dst16 = dbuf[sl]
            w16 = ebuf[sl]
            for f in range(F_PER_TILE):
                g = plsc.load_gather(h_v, [fconsts[f], src16])
                plsc.addupdate_scatter(out_v, [fconsts[f], dst16], g * w16)

    _start(0, srcA, dstA, eeA, semA)

    def _pair(ci, carry):
        _start((2 * ci + 1) * CHUNK, srcB, dstB, eeB, semB)
        _drain(srcA, dstA, eeA, semA)
        _process(srcA, dstA, eeA)
        off_next = jnp.minimum(2 * ci + 2, NCHUNK - 1) * CHUNK
        _start(off_next, srcA, dstA, eeA, semA)
        _drain(srcB, dstB, eeB, semB)
        _process(srcB, dstB, eeB)
        return carry
    lax.fori_loop(0, NCHUNK // 2, _pair, 0)
    _drain(srcA, dstA, eeA, semA)

    @plsc.parallel_loop(0, N_NODES // L, 1, unroll=5)
    def _scale(i):
        sl = pl.ds(i * L, L)
        iv = inv_v[sl]
        for f in range(F_PER_TILE):
            out_v[f, sl] = out_v[f, sl] * iv

    pltpu.sync_copy(out_v, out_hbm.at[w])


def kernel(x, edge_index, W, a_src, a_dst):
    src = edge_index[0]
    dst = edge_index[1]
    s2, d2 = _tc_sd(x, W, jnp.stack([a_src, a_dst], axis=1))
    s = s2.reshape(N_NODES)
    d = d2.reshape(N_NODES)
    h_T = _tc_h(x, W)
    ee, dpart = _sc_edge(src, dst, s, d)
    h_t = h_T.reshape(NW, F_PER_TILE, N_NODES)
    out_t = _sc_spmm(h_t, src, dst, ee, dpart.reshape(NC, N_PAD))
    return out_t.reshape(D, N_NODES).T


# speedup vs baseline: 1.2956x; 1.0077x over previous
import functools

import jax
import jax.numpy as jnp
from jax import lax
from jax.experimental import pallas as pl
from jax.experimental.pallas import tpu as pltpu
from jax.experimental.pallas import tpu_sc as plsc

N_NODES = 10000
N_EDGES = 320000
D = 128

NC = 2
NS = 16
NW = NC * NS
L = 16

N_PAD = 10240
ROWS = N_PAD // L
E_PER_TILE = N_EDGES // NW
CHUNK = 3200
NCHUNK = N_EDGES // CHUNK
UNROLL = 10
F_PER_TILE = D // NW

_mesh = plsc.VectorSubcoreMesh(
    core_axis_name="c", subcore_axis_name="s", num_cores=NC, num_subcores=NS)


def _tc_sd_body(x_ref, w_ref, a2_ref, s_ref, d_ref):
    wa = jnp.dot(w_ref[...], a2_ref[...], preferred_element_type=jnp.float32)
    sd = jnp.dot(x_ref[...], wa, preferred_element_type=jnp.float32)
    s_ref[...] = sd[:, 0:1]
    d_ref[...] = sd[:, 1:2]


def _tc_sd(x, W, a2):
    return pl.pallas_call(
        _tc_sd_body,
        out_shape=[
            jax.ShapeDtypeStruct((N_NODES, 1), jnp.float32),
            jax.ShapeDtypeStruct((N_NODES, 1), jnp.float32),
        ],
    )(x, W, a2)


def _tc_h_body(x_ref, w_ref, h_ref):
    h = jnp.dot(x_ref[...], w_ref[...], preferred_element_type=jnp.float32)
    h_ref[...] = h.T


def _tc_h(x, W):
    return pl.pallas_call(
        _tc_h_body,
        out_shape=jax.ShapeDtypeStruct((D, N_NODES), jnp.float32),
    )(x, W)


@functools.partial(
    pl.kernel,
    out_type=[
        jax.ShapeDtypeStruct((N_EDGES,), jnp.float32),
        jax.ShapeDtypeStruct((NC, ROWS, L), jnp.float32),
    ],
    mesh=_mesh,
    scratch_types=[
        pltpu.VMEM((N_NODES,), jnp.float32),
        pltpu.VMEM((N_NODES,), jnp.float32),
        pltpu.VMEM((E_PER_TILE,), jnp.int32),
        pltpu.VMEM((E_PER_TILE,), jnp.int32),
        pltpu.VMEM((E_PER_TILE,), jnp.float32),
        pltpu.VMEM((ROWS, L), jnp.float32),
        pltpu.VMEM((ROWS,), jnp.int32),
        pltpu.VMEM_SHARED((ROWS, L), jnp.float32),
        pltpu.SemaphoreType.DMA,
    ],
    compiler_params=pltpu.CompilerParams(needs_layout_passes=False, use_tc_tiling_on_sc=False),
)
def _sc_edge(src_hbm, dst_hbm, s_hbm, d_hbm, ee_hbm, dpart_hbm,
             s_v, d_v, src_v, dst_v, ee_v, dloc_v, idx_v, dsh, sem0):
    c = lax.axis_index("c")
    s = lax.axis_index("s")
    w = s * NC + c
    base = w * E_PER_TILE

    pltpu.async_copy(s_hbm, s_v, sem0)
    pltpu.async_copy(d_hbm, d_v, sem0)
    pltpu.async_copy(src_hbm.at[pl.ds(base, E_PER_TILE)], src_v, sem0)
    cp = pltpu.async_copy(dst_hbm.at[pl.ds(base, E_PER_TILE)], dst_v, sem0)
    pltpu.make_async_copy(s_hbm, s_v, sem0).wait()
    pltpu.make_async_copy(d_hbm, d_v, sem0).wait()
    pltpu.make_async_copy(src_hbm.at[pl.ds(base, E_PER_TILE)], src_v,
                          sem0).wait()
    cp.wait()

    zf = jnp.zeros((L,), jnp.float32)
    lane = lax.iota(jnp.int32, L)

    @plsc.parallel_loop(0, ROWS, 1, unroll=8)
    def _zero(i):
        dloc_v[i, :] = zf
        idx_v[pl.ds(i * L, L)] = lane + i * L

    @pl.when(s == 0)
    def _():
        pltpu.sync_copy(dloc_v, dsh)
    plsc.subcore_barrier()

    @plsc.parallel_loop(0, E_PER_TILE // L, 1, unroll=5)
    def _edge(i):
        sl = pl.ds(i * L, L)
        src16 = src_v[sl]
        dst16 = dst_v[sl]
        sv = plsc.load_gather(s_v, [src16])
        dv = plsc.load_gather(d_v, [dst16])
        e = sv + dv
        e = jnp.maximum(e, 0.2 * e)
        ee = jnp.exp(e)
        ee_v[sl] = ee
        plsc.addupdate_scatter(
            dloc_v, [lax.shift_right_logical(dst16, 4), dst16 & 15], ee)

    pltpu.sync_copy(ee_v, ee_hbm.at[pl.ds(base, E_PER_TILE)])

    pltpu.sync_copy(dloc_v, dsh.at[idx_v], add=True)
    plsc.subcore_barrier()

    @pl.when(s == 0)
    def _():
        pltpu.sync_copy(dsh, dloc_v)
        pltpu.sync_copy(dloc_v, dpart_hbm.at[c])


@functools.partial(
    pl.kernel,
    out_type=jax.ShapeDtypeStruct((NW, F_PER_TILE, N_NODES), jnp.float32),
    mesh=_mesh,
    scratch_types=[
        pltpu.VMEM((F_PER_TILE, N_NODES), jnp.float32),
        pltpu.VMEM((F_PER_TILE, N_NODES), jnp.float32),
        pltpu.VMEM((N_PAD,), jnp.float32),
        pltpu.VMEM((N_PAD,), jnp.float32),
        pltpu.VMEM((CHUNK,), jnp.int32),
        pltpu.VMEM((CHUNK,), jnp.int32),
        pltpu.VMEM((CHUNK,), jnp.float32),
        pltpu.VMEM((CHUNK,), jnp.int32),
        pltpu.VMEM((CHUNK,), jnp.int32),
        pltpu.VMEM((CHUNK,), jnp.float32),
        pltpu.SemaphoreType.DMA,
        pltpu.SemaphoreType.DMA,
    ],
    compiler_params=pltpu.CompilerParams(needs_layout_passes=False, use_tc_tiling_on_sc=False),
)
def _sc_spmm(ht_hbm, src_hbm, dst_hbm, ee_hbm, dpart_hbm, out_hbm,
             h_v, out_v, inv_v, tmp_v, srcA, dstA, eeA, srcB, dstB, eeB,
             semA, semB):
    c = lax.axis_index("c")
    s = lax.axis_index("s")
    w = s * NC + c

    pltpu.async_copy(ht_hbm.at[w], h_v, semA)
    pltpu.async_copy(dpart_hbm.at[0], inv_v, semA)
    pltpu.async_copy(dpart_hbm.at[1], tmp_v, semA)
    pltpu.make_async_copy(ht_hbm.at[w], h_v, semA).wait()
    pltpu.make_async_copy(dpart_hbm.at[0], inv_v, semA).wait()
    pltpu.make_async_copy(dpart_hbm.at[1], tmp_v, semA).wait()

    zf = jnp.zeros((L,), jnp.float32)

    @plsc.parallel_loop(0, ROWS, 1, unroll=8)
    def _prep(i):
        sl = pl.ds(i * L, L)
        t = inv_v[sl] + tmp_v[sl]
        inv_v[sl] = 1.0 / (t + 1e-16)

    for f in range(F_PER_TILE):
        @plsc.parallel_loop(0, N_NODES // L, 1, unroll=5)
        def _zero(i, f=f):
            out_v[f, pl.ds(i * L, L)] = zf

    fconsts = [jnp.full((L,), f, jnp.int32) for f in range(F_PER_TILE)]

    def _start(off, sbuf, dbuf, ebuf, sem):
        pltpu.async_copy(src_hbm.at[pl.ds(off, CHUNK)], sbuf, sem)
        pltpu.async_copy(dst_hbm.at[pl.ds(off, CHUNK)], dbuf, sem)
        pltpu.async_copy(ee_hbm.at[pl.ds(off, CHUNK)], ebuf, sem)

    def _drain(sbuf, dbuf, ebuf, sem):
        pltpu.make_async_copy(src_hbm.at[pl.ds(0, CHUNK)], sbuf, sem).wait()
        pltpu.make_async_copy(dst_hbm.at[pl.ds(0, CHUNK)], dbuf, sem).wait()
        pltpu.make_async_copy(ee_hbm.at[pl.ds(0, CHUNK)], ebuf, sem).wait()

    def _process(sbuf, dbuf, ebuf):
        @plsc.parallel_loop(0, CHUNK // L, 1, unroll=UNROLL)
        def _grp(i):
            sl = pl.ds(i * L, L)
            src16 = sbuf[sl]
            dst16 = dbuf[sl]
            w16 = ebuf[sl]
            for f in range(F_PER_TILE):
                g = plsc.load_gather(h_v, [fconsts[f], src16])
                plsc.addupdate_scatter(out_v, [fconsts[f], dst16], g * w16)

    _start(0, srcA, dstA, eeA, semA)

    def _pair(ci, carry):
        _start((2 * ci + 1) * CHUNK, srcB, dstB, eeB, semB)
        _drain(srcA, dstA, eeA, semA)
        _process(srcA, dstA, eeA)
        off_next = jnp.minimum(2 * ci + 2, NCHUNK - 1) * CHUNK
        _start(off_next, srcA, dstA, eeA, semA)
        _drain(srcB, dstB, eeB, semB)
        _process(srcB, dstB, eeB)
        return carry
    lax.fori_loop(0, NCHUNK // 2, _pair, 0)
    _drain(srcA, dstA, eeA, semA)

    @plsc.parallel_loop(0, N_NODES // L, 1, unroll=5)
    def _scale(i):
        sl = pl.ds(i * L, L)
        iv = inv_v[sl]
        for f in range(F_PER_TILE):
            out_v[f, sl] = out_v[f, sl] * iv

    pltpu.sync_copy(out_v, out_hbm.at[w])


def kernel(x, edge_index, W, a_src, a_dst):
    src = edge_index[0]
    dst = edge_index[1]
    s2, d2 = _tc_sd(x, W, jnp.stack([a_src, a_dst], axis=1))
    s = s2.reshape(N_NODES)
    d = d2.reshape(N_NODES)
    h_T = _tc_h(x, W)
    ee, dpart = _sc_edge(src, dst, s, d)
    h_t = h_T.reshape(NW, F_PER_TILE, N_NODES)
    out_t = _sc_spmm(h_t, src, dst, ee, dpart.reshape(NC, N_PAD))
    return out_t.reshape(D, N_NODES).T
